# Initial kernel scaffold; baseline (speedup 1.0000x reference)
import jax
import jax.numpy as jnp
from jax.experimental import pallas as pl


def kernel(z, embedding):
    raise NotImplementedError("write your pallas kernel here")


# speedup vs baseline: 9.7439x; 9.7439x over previous
import functools

import jax
import jax.numpy as jnp
from jax import lax
from jax.experimental import pallas as pl
from jax.experimental.pallas import tpu as pltpu
from jax.experimental.pallas import tpu_sc as plsc

N_BINS = 8192
D_MODEL = 256
N_TOK = 16384
M_BLK = 256
N_ELEMS = N_TOK * D_MODEL

CHUNK = 2736
CPAD = 2816
NCHUNK = 3


def _requant(v):
    return v.astype(jnp.bfloat16).astype(jnp.float32)


def _argmin_body(z_ref, zs_ref, embT_ref, es_ref, idx_ref, loss_ref):
    i = pl.program_id(0)
    nsteps = pl.num_programs(0)

    z = z_ref[...]
    zs = zs_ref[...]
    zb = (-2.0 * z).astype(jnp.bfloat16)

    acc_q = None
    for c in range(NCHUNK):
        e_c = embT_ref[:, c * CPAD:(c + 1) * CPAD]
        es_c = es_ref[:, c * CPAD:(c + 1) * CPAD]
        mm2 = jnp.dot(zb, e_c, preferred_element_type=jnp.float32)
        dist = (zs + es_c) + mm2
        v = jnp.min(dist, axis=1, keepdims=True)
        kio = lax.broadcasted_iota(jnp.int32, dist.shape, 1)
        li = jnp.min(jnp.where(dist == v, kio, CPAD), axis=1, keepdims=True)
        gi = li + c * CHUNK
        if c == 0:
            acc_q, acc_i, lmin = _requant(v), gi, v
        else:
            upd = v < acc_q
            acc_i = jnp.where(upd, gi, acc_i)
            acc_q = jnp.where(upd, _requant(v), acc_q)
            lmin = jnp.minimum(lmin, v)

    idx_ref[0, 0, :] = acc_i[:, 0]

    @pl.when(i == 0)
    def _():
        loss_ref[...] = jnp.zeros_like(loss_ref)

    loss_ref[...] += jnp.sum(lmin).reshape(1, 1)

    @pl.when(i == nsteps - 1)
    def _():
        loss_ref[...] *= jnp.float32(1.25 / N_ELEMS)


def _argmin_loss(z2d, zs2d, embT16, es_pad):
    grid = N_TOK // M_BLK
    return pl.pallas_call(
        _argmin_body,
        grid=(grid,),
        in_specs=[
            pl.BlockSpec((M_BLK, D_MODEL), lambda i: (i, 0)),
            pl.BlockSpec((M_BLK, 1), lambda i: (i, 0)),
            pl.BlockSpec((D_MODEL, NCHUNK * CPAD), lambda i: (0, 0)),
            pl.BlockSpec((1, NCHUNK * CPAD), lambda i: (0, 0)),
        ],
        out_specs=[
            pl.BlockSpec((1, 1, M_BLK), lambda i: (i, 0, 0)),
            pl.BlockSpec((1, 1), lambda i: (0, 0)),
        ],
        out_shape=[
            jax.ShapeDtypeStruct((grid, 1, M_BLK), jnp.int32),
            jax.ShapeDtypeStruct((1, 1), jnp.float32),
        ],
        compiler_params=pltpu.CompilerParams(
            dimension_semantics=("arbitrary",),
        ),
    )(z2d, zs2d, embT16, es_pad)


_NW = 32
_B_PER_W = N_TOK // _NW
_CHUNK_G = 128


def _sc_gather(embedding, idx_flat):
    mesh = plsc.VectorSubcoreMesh(core_axis_name="c", subcore_axis_name="s")

    @functools.partial(
        pl.kernel,
        mesh=mesh,
        out_type=jax.ShapeDtypeStruct((N_TOK, D_MODEL), jnp.float32),
        scratch_types=[
            pltpu.VMEM((_B_PER_W,), jnp.int32),
            pltpu.VMEM((_CHUNK_G, D_MODEL), jnp.float32),
            pltpu.SemaphoreType.DMA,
        ],
    )
    def k(table_hbm, idx_hbm, out_hbm, idx_v, rows_v, sem):
        wid = lax.axis_index("s") * 2 + lax.axis_index("c")
        base = wid * _B_PER_W
        pltpu.sync_copy(idx_hbm.at[pl.ds(base, _B_PER_W)], idx_v)

        @pl.loop(0, _B_PER_W, step=_CHUNK_G)
        def _(c):
            pltpu.async_copy(
                table_hbm.at[idx_v.at[pl.ds(c, _CHUNK_G)]], rows_v, sem
            ).wait()
            pltpu.sync_copy(rows_v, out_hbm.at[pl.ds(base + c, _CHUNK_G)])

    return k(embedding, idx_flat)


def kernel(z, embedding):
    B, T, d = z.shape
    z2d = z.reshape(N_TOK, d)
    zs2d = jnp.sum(z2d ** 2, axis=1, keepdims=True)
    es = jnp.sum(embedding ** 2, axis=1)

    embT16 = embedding.T.astype(jnp.bfloat16)
    epad = jnp.zeros((D_MODEL, NCHUNK, CPAD), jnp.bfloat16)
    es_pad = jnp.full((NCHUNK, CPAD), jnp.inf, jnp.float32)
    for c in range(NCHUNK):
        lo = c * CHUNK
        w = min(CHUNK, N_BINS - lo)
        epad = epad.at[:, c, :w].set(embT16[:, lo:lo + w])
        es_pad = es_pad.at[c, :w].set(es[lo:lo + w])
    epad = epad.reshape(D_MODEL, NCHUNK * CPAD)
    es_pad = es_pad.reshape(1, NCHUNK * CPAD)

    idx_blocks, loss2d = _argmin_loss(z2d, zs2d, epad, es_pad)
    idx_flat = idx_blocks.reshape(N_TOK)
    z_q = _sc_gather(embedding, idx_flat)
    z_q_st = z_q.reshape(z.shape)
    loss = loss2d.reshape(())
    indices = idx_flat.reshape(B, T)
    return (z_q_st, loss, indices)


# speedup vs baseline: 10.5990x; 1.0878x over previous
import functools

import jax
import jax.numpy as jnp
from jax import lax
from jax.experimental import pallas as pl
from jax.experimental.pallas import tpu as pltpu
from jax.experimental.pallas import tpu_sc as plsc

N_BINS = 8192
D_MODEL = 256
N_TOK = 16384
M_BLK = 512
N_ELEMS = N_TOK * D_MODEL

CHUNK = 2736
CPAD = 2816
NCHUNK = 3


def _requant(v):
    return v.astype(jnp.bfloat16).astype(jnp.float32)


def _argmin_body(z_ref, zs_ref, embT_ref, es_ref, idx_ref, loss_ref):
    i = pl.program_id(0)
    nsteps = pl.num_programs(0)

    z = z_ref[...]
    zs = zs_ref[...]
    zb = (-2.0 * z).astype(jnp.bfloat16)

    acc_q = None
    for c in range(NCHUNK):
        e_c = embT_ref[:, c * CPAD:(c + 1) * CPAD]
        es_c = es_ref[:, c * CPAD:(c + 1) * CPAD]
        mm2 = jnp.dot(zb, e_c, preferred_element_type=jnp.float32)
        dist = (zs + es_c) + mm2
        v = jnp.min(dist, axis=1, keepdims=True)
        kio = lax.broadcasted_iota(jnp.int32, dist.shape, 1)
        li = jnp.min(jnp.where(dist == v, kio, CPAD), axis=1, keepdims=True)
        gi = li + c * CHUNK
        if c == 0:
            acc_q, acc_i, lmin = _requant(v), gi, v
        else:
            upd = v < acc_q
            acc_i = jnp.where(upd, gi, acc_i)
            acc_q = jnp.where(upd, _requant(v), acc_q)
            lmin = jnp.minimum(lmin, v)

    idx_ref[0, 0, :] = acc_i[:, 0]

    @pl.when(i == 0)
    def _():
        loss_ref[...] = jnp.zeros_like(loss_ref)

    loss_ref[...] += jnp.sum(lmin).reshape(1, 1)

    @pl.when(i == nsteps - 1)
    def _():
        loss_ref[...] *= jnp.float32(1.25 / N_ELEMS)


def _argmin_loss(z2d, zs2d, embT16, es_pad):
    grid = N_TOK // M_BLK
    return pl.pallas_call(
        _argmin_body,
        grid=(grid,),
        in_specs=[
            pl.BlockSpec((M_BLK, D_MODEL), lambda i: (i, 0)),
            pl.BlockSpec((M_BLK, 1), lambda i: (i, 0)),
            pl.BlockSpec((D_MODEL, NCHUNK * CPAD), lambda i: (0, 0)),
            pl.BlockSpec((1, NCHUNK * CPAD), lambda i: (0, 0)),
        ],
        out_specs=[
            pl.BlockSpec((1, 1, M_BLK), lambda i: (i, 0, 0)),
            pl.BlockSpec((1, 1), lambda i: (0, 0)),
        ],
        out_shape=[
            jax.ShapeDtypeStruct((grid, 1, M_BLK), jnp.int32),
            jax.ShapeDtypeStruct((1, 1), jnp.float32),
        ],
        compiler_params=pltpu.CompilerParams(
            dimension_semantics=("arbitrary",),
        ),
    )(z2d, zs2d, embT16, es_pad)


_NW = 32
_B_PER_W = N_TOK // _NW
_CHUNK_G = 128


def _sc_gather(embedding, idx_flat):
    mesh = plsc.VectorSubcoreMesh(core_axis_name="c", subcore_axis_name="s")

    @functools.partial(
        pl.kernel,
        mesh=mesh,
        out_type=jax.ShapeDtypeStruct((N_TOK, D_MODEL), jnp.float32),
        scratch_types=[
            pltpu.VMEM((_B_PER_W,), jnp.int32),
            pltpu.VMEM((_CHUNK_G, D_MODEL), jnp.float32),
            pltpu.SemaphoreType.DMA,
        ],
    )
    def k(table_hbm, idx_hbm, out_hbm, idx_v, rows_v, sem):
        wid = lax.axis_index("s") * 2 + lax.axis_index("c")
        base = wid * _B_PER_W
        pltpu.sync_copy(idx_hbm.at[pl.ds(base, _B_PER_W)], idx_v)

        @pl.loop(0, _B_PER_W, step=_CHUNK_G)
        def _(c):
            pltpu.async_copy(
                table_hbm.at[idx_v.at[pl.ds(c, _CHUNK_G)]], rows_v, sem
            ).wait()
            pltpu.sync_copy(rows_v, out_hbm.at[pl.ds(base + c, _CHUNK_G)])

    return k(embedding, idx_flat)


def kernel(z, embedding):
    B, T, d = z.shape
    z2d = z.reshape(N_TOK, d)
    zs2d = jnp.sum(z2d ** 2, axis=1, keepdims=True)
    es = jnp.sum(embedding ** 2, axis=1)

    embT16 = embedding.T.astype(jnp.bfloat16)
    epad = jnp.zeros((D_MODEL, NCHUNK, CPAD), jnp.bfloat16)
    es_pad = jnp.full((NCHUNK, CPAD), jnp.inf, jnp.float32)
    for c in range(NCHUNK):
        lo = c * CHUNK
        w = min(CHUNK, N_BINS - lo)
        epad = epad.at[:, c, :w].set(embT16[:, lo:lo + w])
        es_pad = es_pad.at[c, :w].set(es[lo:lo + w])
    epad = epad.reshape(D_MODEL, NCHUNK * CPAD)
    es_pad = es_pad.reshape(1, NCHUNK * CPAD)

    idx_blocks, loss2d = _argmin_loss(z2d, zs2d, epad, es_pad)
    idx_flat = idx_blocks.reshape(N_TOK)
    z_q = _sc_gather(embedding, idx_flat)
    z_q_st = z_q.reshape(z.shape)
    loss = loss2d.reshape(())
    indices = idx_flat.reshape(B, T)
    return (z_q_st, loss, indices)


# speedup vs baseline: 13.5010x; 1.2738x over previous
import functools

import jax
import jax.numpy as jnp
from jax import lax
from jax.experimental import pallas as pl
from jax.experimental.pallas import tpu as pltpu
from jax.experimental.pallas import tpu_sc as plsc

N_BINS = 8192
D_MODEL = 256
N_TOK = 16384
M_BLK = 512
N_ELEMS = N_TOK * D_MODEL

CHUNK = 2736
CPAD = 2816
NCHUNK = 3


def _requant(v):
    return v.astype(jnp.bfloat16).astype(jnp.float32)


def _argmin_body(z_ref, zs_ref, emb_ref, es_ref, idx_ref, loss_ref):
    i = pl.program_id(0)
    nsteps = pl.num_programs(0)

    z = z_ref[...]
    zs = zs_ref[...]
    zb = (-2.0 * z).astype(jnp.bfloat16)
    lane = lax.broadcasted_iota(jnp.int32, (M_BLK, 128), 1)

    acc_q = None
    for c in range(NCHUNK):
        e_c = emb_ref[c * CPAD:(c + 1) * CPAD, :]
        mm2 = lax.dot_general(zb, e_c, (((1,), (1,)), ((), ())),
                              preferred_element_type=jnp.float32)
        vs, ss = [], []
        for j in range(CPAD // 128):
            es_j = es_ref[:, c * CPAD + j * 128:c * CPAD + (j + 1) * 128]
            vs.append((zs + es_j) + mm2[:, j * 128:(j + 1) * 128])
            ss.append(jnp.full((M_BLK, 128), j, jnp.int32))
        while len(vs) > 1:
            nv, ns = [], []
            for a in range(0, len(vs) - 1, 2):
                le = vs[a] <= vs[a + 1]
                nv.append(jnp.minimum(vs[a], vs[a + 1]))
                ns.append(jnp.where(le, ss[a], ss[a + 1]))
            if len(vs) % 2:
                nv.append(vs[-1])
                ns.append(ss[-1])
            vs, ss = nv, ns
        vfin, sfin = vs[0], ss[0]
        v = jnp.min(vfin, axis=1, keepdims=True)
        ifin = sfin * 128 + lane
        li = jnp.min(jnp.where(vfin == v, ifin, CPAD),
                     axis=1, keepdims=True)
        gi = li + c * CHUNK
        if c == 0:
            acc_q, acc_i, lmin = _requant(v), gi, v
        else:
            upd = v < acc_q
            acc_i = jnp.where(upd, gi, acc_i)
            acc_q = jnp.where(upd, _requant(v), acc_q)
            lmin = jnp.minimum(lmin, v)

    idx_ref[0, 0, :] = acc_i[:, 0]

    @pl.when(i == 0)
    def _():
        loss_ref[...] = jnp.zeros_like(loss_ref)

    loss_ref[...] += jnp.sum(lmin).reshape(1, 1)

    @pl.when(i == nsteps - 1)
    def _():
        loss_ref[...] *= jnp.float32(1.25 / N_ELEMS)


def _argmin_loss(z2d, zs2d, emb16_pad, es_pad):
    grid = N_TOK // M_BLK
    return pl.pallas_call(
        _argmin_body,
        grid=(grid,),
        in_specs=[
            pl.BlockSpec((M_BLK, D_MODEL), lambda i: (i, 0)),
            pl.BlockSpec((M_BLK, 1), lambda i: (i, 0)),
            pl.BlockSpec((NCHUNK * CPAD, D_MODEL), lambda i: (0, 0)),
            pl.BlockSpec((1, NCHUNK * CPAD), lambda i: (0, 0)),
        ],
        out_specs=[
            pl.BlockSpec((1, 1, M_BLK), lambda i: (i, 0, 0)),
            pl.BlockSpec((1, 1), lambda i: (0, 0)),
        ],
        out_shape=[
            jax.ShapeDtypeStruct((grid, 1, M_BLK), jnp.int32),
            jax.ShapeDtypeStruct((1, 1), jnp.float32),
        ],
        compiler_params=pltpu.CompilerParams(
            dimension_semantics=("arbitrary",),
        ),
    )(z2d, zs2d, emb16_pad, es_pad)


_NW = 32
_B_PER_W = N_TOK // _NW
_CHUNK_G = 128


def _sc_gather(embedding, idx_flat):
    mesh = plsc.VectorSubcoreMesh(core_axis_name="c", subcore_axis_name="s")

    @functools.partial(
        pl.kernel,
        mesh=mesh,
        out_type=jax.ShapeDtypeStruct((N_TOK, D_MODEL), jnp.float32),
        scratch_types=[
            pltpu.VMEM((_B_PER_W,), jnp.int32),
            pltpu.VMEM((_CHUNK_G, D_MODEL), jnp.float32),
            pltpu.SemaphoreType.DMA,
        ],
    )
    def k(table_hbm, idx_hbm, out_hbm, idx_v, rows_v, sem):
        wid = lax.axis_index("s") * 2 + lax.axis_index("c")
        base = wid * _B_PER_W
        pltpu.sync_copy(idx_hbm.at[pl.ds(base, _B_PER_W)], idx_v)

        @pl.loop(0, _B_PER_W, step=_CHUNK_G)
        def _(c):
            pltpu.async_copy(
                table_hbm.at[idx_v.at[pl.ds(c, _CHUNK_G)]], rows_v, sem
            ).wait()
            pltpu.sync_copy(rows_v, out_hbm.at[pl.ds(base + c, _CHUNK_G)])

    return k(embedding, idx_flat)


def kernel(z, embedding):
    B, T, d = z.shape
    z2d = z.reshape(N_TOK, d)
    zs2d = jnp.sum(z2d ** 2, axis=1, keepdims=True)
    es = jnp.sum(embedding ** 2, axis=1)

    emb16 = embedding.astype(jnp.bfloat16)
    zrow = jnp.zeros((CPAD - CHUNK, D_MODEL), jnp.bfloat16)
    ztail = jnp.zeros((CPAD - (N_BINS - 2 * CHUNK), D_MODEL), jnp.bfloat16)
    emb16_pad = jnp.concatenate(
        [emb16[:CHUNK], zrow, emb16[CHUNK:2 * CHUNK], zrow,
         emb16[2 * CHUNK:], ztail], axis=0)
    inf_pad = jnp.full((CPAD - CHUNK,), jnp.inf, jnp.float32)
    inf_tail = jnp.full((CPAD - (N_BINS - 2 * CHUNK),), jnp.inf, jnp.float32)
    es_pad = jnp.concatenate(
        [es[:CHUNK], inf_pad, es[CHUNK:2 * CHUNK], inf_pad,
         es[2 * CHUNK:], inf_tail]).reshape(1, NCHUNK * CPAD)

    idx_blocks, loss2d = _argmin_loss(z2d, zs2d, emb16_pad, es_pad)
    idx_flat = idx_blocks.reshape(N_TOK)
    z_q = _sc_gather(embedding, idx_flat)
    z_q_st = z_q.reshape(z.shape)
    loss = loss2d.reshape(())
    indices = idx_flat.reshape(B, T)
    return (z_q_st, loss, indices)
